# baseline (speedup 1.0000x reference)
import functools

import jax
import jax.numpy as jnp
from jax import lax
from jax.experimental import pallas as pl
from jax.experimental.pallas import tpu as pltpu
from jax.experimental.pallas import tpu_sc as plsc

B, H, W, D = 16, 32, 32, 512
HW = H * W
NW = 32
TOK = B * HW
TPW = TOK // NW
T = 64
NCHUNK = TPW // T
L = 16


def _pos_embed_planes():
    d_half = D // 4

    def get_1d(n, dh):
        omega = jnp.arange(dh).astype(jnp.float32) / dh
        omega = 1.0 / (10000.0 ** omega)
        p = jnp.arange(n).astype(jnp.float32)
        out = jnp.einsum('n,d->nd', p, omega)
        return jnp.stack([jnp.sin(out), jnp.cos(out)], axis=-1).reshape(n, -1)

    pe_h = get_1d(H, d_half)
    pe_w = get_1d(W, d_half)
    pe = jnp.concatenate([
        jnp.repeat(pe_h[:, None, :], W, axis=1),
        jnp.repeat(pe_w[None, :, :], H, axis=0),
    ], axis=-1)
    pos_t = pe.reshape(HW, D).T
    return pos_t.reshape(D, HW // T, T).transpose(1, 0, 2)


_mesh = plsc.VectorSubcoreMesh(
    core_axis_name="c", subcore_axis_name="s", num_cores=2, num_subcores=16)


@functools.partial(
    pl.kernel,
    out_type=jax.ShapeDtypeStruct((B, D, HW), jnp.float32),
    mesh=_mesh,
    scratch_types=[
        pltpu.VMEM((T,), jnp.int32),
        pltpu.VMEM((T,), jnp.int32),
        pltpu.VMEM((T, D), jnp.float32),
        pltpu.VMEM((T, D), jnp.float32),
        pltpu.VMEM((D, T), jnp.float32),
        pltpu.SemaphoreType.DMA,
        pltpu.SemaphoreType.DMA,
    ],
    compiler_params=pltpu.CompilerParams(
        use_tc_tiling_on_sc=False, needs_layout_passes=False),
)
def _emb_kernel(y1_hbm, y2_hbm, w1_hbm, w2_hbm, pc_hbm, out_hbm,
                idx1_v, idx2_v, rows1_v, rows2_v, outt_v, sem1, sem2):
    wid = lax.axis_index("s") * 2 + lax.axis_index("c")
    b = wid // 2
    half = wid % 2
    base = wid * TPW

    iota = lax.iota(jnp.int32, L)

    for c in range(NCHUNK):
        cb = base + c * T
        c2 = half * NCHUNK + c
        pltpu.sync_copy(y1_hbm.at[pl.ds(cb, T)], idx1_v)
        pltpu.sync_copy(y2_hbm.at[pl.ds(cb, T)], idx2_v)
        d1 = pltpu.async_copy(w1_hbm.at[idx1_v], rows1_v, sem1)
        d2 = pltpu.async_copy(w2_hbm.at[idx2_v], rows2_v, sem2)
        pltpu.sync_copy(pc_hbm.at[c2], outt_v)
        d1.wait()
        d2.wait()

        def body(d, _):
            cvec = jnp.full((L,), d, dtype=jnp.int32)
            for rg in range(T // L):
                ridx = iota + (rg * L)
                g1 = plsc.load_gather(rows1_v, [ridx, cvec])
                g2 = plsc.load_gather(rows2_v, [ridx, cvec])
                plsc.addupdate_scatter(outt_v, [cvec, ridx], g1 + g2)
            return _

        lax.fori_loop(0, D, body, None)

        pltpu.sync_copy(outt_v, out_hbm.at[b, :, pl.ds(half * (HW // 2) + c * T, T)])


def kernel(y1_idx, y2_idx, W1, W2):
    pc = _pos_embed_planes()
    y1f = y1_idx.reshape(TOK).astype(jnp.int32)
    y2f = y2_idx.reshape(TOK).astype(jnp.int32)
    out = _emb_kernel(y1f, y2f, W1, W2, pc)
    return out.reshape(B, D, H, W)


# speedup vs baseline: 1.4458x; 1.4458x over previous
import functools

import jax
import jax.numpy as jnp
from jax import lax
from jax.experimental import pallas as pl
from jax.experimental.pallas import tpu as pltpu
from jax.experimental.pallas import tpu_sc as plsc

B, H, W, D = 16, 32, 32, 512
HW = H * W
NW = 32
TOK = B * HW
TPW = TOK // NW
T = 128
NCHUNK = TPW // T
DH = D // 2
L = 16


def _pos_embed_planes():
    d_half = D // 4

    def get_1d(n, dh):
        omega = jnp.arange(dh).astype(jnp.float32) / dh
        omega = 1.0 / (10000.0 ** omega)
        p = jnp.arange(n).astype(jnp.float32)
        out = jnp.einsum('n,d->nd', p, omega)
        return jnp.stack([jnp.sin(out), jnp.cos(out)], axis=-1).reshape(n, -1)

    pe_h = get_1d(H, d_half)
    pe_w = get_1d(W, d_half)
    pe = jnp.concatenate([
        jnp.repeat(pe_h[:, None, :], W, axis=1),
        jnp.repeat(pe_w[None, :, :], H, axis=0),
    ], axis=-1)
    pos_t = pe.reshape(HW, D).T
    return pos_t.reshape(2, DH, HW // T, T).transpose(2, 0, 1, 3)


_mesh = plsc.VectorSubcoreMesh(
    core_axis_name="c", subcore_axis_name="s", num_cores=2, num_subcores=16)


@functools.partial(
    pl.kernel,
    out_type=jax.ShapeDtypeStruct((B, D, HW), jnp.float32),
    mesh=_mesh,
    scratch_types=[
        pltpu.VMEM((T,), jnp.int32),
        pltpu.VMEM((T,), jnp.int32),
        pltpu.VMEM((T, DH), jnp.float32),
        pltpu.VMEM((T, DH), jnp.float32),
        pltpu.VMEM((DH, T), jnp.float32),
        pltpu.SemaphoreType.DMA,
        pltpu.SemaphoreType.DMA,
    ],
    compiler_params=pltpu.CompilerParams(needs_layout_passes=False),
)
def _emb_kernel(y1_hbm, y2_hbm, w1_hbm, w2_hbm, pc_hbm, out_hbm,
                idx1_v, idx2_v, rows1_v, rows2_v, outt_v, sem1, sem2):
    wid = lax.axis_index("s") * 2 + lax.axis_index("c")
    b = wid // 2
    half = wid % 2
    base = wid * TPW

    iota = lax.iota(jnp.int32, L)

    for c in range(NCHUNK):
        cb = base + c * T
        c2 = half * NCHUNK + c
        hw0 = half * (HW // 2) + c * T
        pltpu.sync_copy(y1_hbm.at[pl.ds(cb, T)], idx1_v)
        pltpu.sync_copy(y2_hbm.at[pl.ds(cb, T)], idx2_v)

        for dh in range(2):
            d1 = pltpu.async_copy(
                w1_hbm.at[idx1_v, pl.ds(dh * DH, DH)], rows1_v, sem1)
            d2 = pltpu.async_copy(
                w2_hbm.at[idx2_v, pl.ds(dh * DH, DH)], rows2_v, sem2)
            pltpu.sync_copy(pc_hbm.at[c2, dh], outt_v)
            d1.wait()
            d2.wait()

            @plsc.parallel_loop(0, DH, 1)
            def _(d):
                cvec = jnp.full((L,), d, dtype=jnp.int32)
                for rg in range(T // L):
                    ridx = iota + (rg * L)
                    g1 = plsc.load_gather(rows1_v, [ridx, cvec])
                    g2 = plsc.load_gather(rows2_v, [ridx, cvec])
                    plsc.addupdate_scatter(outt_v, [cvec, ridx], g1 + g2)

            pltpu.sync_copy(
                outt_v, out_hbm.at[b, pl.ds(dh * DH, DH), pl.ds(hw0, T)])


def kernel(y1_idx, y2_idx, W1, W2):
    pc = _pos_embed_planes()
    y1f = y1_idx.reshape(TOK).astype(jnp.int32)
    y2f = y2_idx.reshape(TOK).astype(jnp.int32)
    out = _emb_kernel(y1f, y2f, W1, W2, pc)
    return out.reshape(B, D, H, W)


# speedup vs baseline: 3.3360x; 2.3074x over previous
import functools

import jax
import jax.numpy as jnp
from jax import lax
from jax.experimental import pallas as pl
from jax.experimental.pallas import tpu as pltpu
from jax.experimental.pallas import tpu_sc as plsc

B, H, W, D = 16, 32, 32, 512
HW = H * W
NW = 32
TOK = B * HW
TPW = TOK // NW
T = 128
NCHUNK = TPW // T
DH = D // 2
L = 16


def _pos_embed_planes():
    d_half = D // 4

    def get_1d(n, dh):
        omega = jnp.arange(dh).astype(jnp.float32) / dh
        omega = 1.0 / (10000.0 ** omega)
        p = jnp.arange(n).astype(jnp.float32)
        out = jnp.einsum('n,d->nd', p, omega)
        return jnp.stack([jnp.sin(out), jnp.cos(out)], axis=-1).reshape(n, -1)

    pe_h = get_1d(H, d_half)
    pe_w = get_1d(W, d_half)
    pe = jnp.concatenate([
        jnp.repeat(pe_h[:, None, :], W, axis=1),
        jnp.repeat(pe_w[None, :, :], H, axis=0),
    ], axis=-1)
    pos_t = pe.reshape(HW, D).T
    return pos_t.reshape(2, DH, HW // T, T).transpose(2, 0, 1, 3)


_mesh = plsc.VectorSubcoreMesh(
    core_axis_name="c", subcore_axis_name="s", num_cores=2, num_subcores=16)


@functools.partial(
    pl.kernel,
    out_type=jax.ShapeDtypeStruct((B, D, HW), jnp.float32),
    mesh=_mesh,
    scratch_types=[
        pltpu.VMEM((T,), jnp.int32),
        pltpu.VMEM((T,), jnp.int32),
        pltpu.VMEM((T, DH), jnp.float32),
        pltpu.VMEM((T, DH), jnp.float32),
        pltpu.VMEM((DH, T), jnp.float32),
        pltpu.SemaphoreType.DMA,
        pltpu.SemaphoreType.DMA,
    ],
    compiler_params=pltpu.CompilerParams(needs_layout_passes=False),
)
def _emb_kernel(y1_hbm, y2_hbm, w1_hbm, w2_hbm, pc_hbm, out_hbm,
                idx1_v, idx2_v, rows1_v, rows2_v, outt_v, sem1, sem2):
    wid = lax.axis_index("s") * 2 + lax.axis_index("c")
    b = wid // 2
    half = wid % 2
    base = wid * TPW

    iota = lax.iota(jnp.int32, L)
    diag = [(iota + k) & (L - 1) for k in range(L)]

    for c in range(NCHUNK):
        cb = base + c * T
        c2 = half * NCHUNK + c
        hw0 = half * (HW // 2) + c * T
        pltpu.sync_copy(y1_hbm.at[pl.ds(cb, T)], idx1_v)
        pltpu.sync_copy(y2_hbm.at[pl.ds(cb, T)], idx2_v)

        for dh in range(2):
            d1 = pltpu.async_copy(
                w1_hbm.at[idx1_v, pl.ds(dh * DH, DH)], rows1_v, sem1)
            d2 = pltpu.async_copy(
                w2_hbm.at[idx2_v, pl.ds(dh * DH, DH)], rows2_v, sem2)
            pltpu.sync_copy(pc_hbm.at[c2, dh], outt_v)
            d1.wait()
            d2.wait()

            @plsc.parallel_loop(0, (DH // L) * (T // L), 1)
            def _(blk):
                d0 = (blk >> 3) * L
                r0 = (blk & 7) * L
                dvec = jnp.full((L,), d0, dtype=jnp.int32)
                rvec = jnp.full((L,), r0, dtype=jnp.int32) + iota
                for k in range(L):
                    cvec = dvec + diag[k]
                    g1 = plsc.load_gather(rows1_v, [rvec, cvec])
                    g2 = plsc.load_gather(rows2_v, [rvec, cvec])
                    plsc.addupdate_scatter(outt_v, [cvec, rvec], g1 + g2)

            pltpu.sync_copy(
                outt_v, out_hbm.at[b, pl.ds(dh * DH, DH), pl.ds(hw0, T)])


def kernel(y1_idx, y2_idx, W1, W2):
    pc = _pos_embed_planes()
    y1f = y1_idx.reshape(TOK).astype(jnp.int32)
    y2f = y2_idx.reshape(TOK).astype(jnp.int32)
    out = _emb_kernel(y1f, y2f, W1, W2, pc)
    return out.reshape(B, D, H, W)
